# baseline (speedup 1.0000x reference)
import functools
import math

import jax
import jax.numpy as jnp
from jax import lax
from jax.experimental import pallas as pl
from jax.experimental.pallas import tpu as pltpu
from jax.experimental.pallas import tpu_sc as plsc

B, S, D = 4, 2048, 2048
H, HD = 16, 128
HIDDEN = 5632
CAP = 0.12
EPS = 1e-5
K = int(S * CAP)
KP = 256
NPAD = KP - K
BS = B * S
BK = B * KP
NEG = float(jnp.finfo(jnp.float32).min)

_NC, _NS = 2, 16
_NW = _NC * _NS
_GPW = BK // _NW
_CPW = BS // _NW

_sc_mesh = plsc.VectorSubcoreMesh(
    core_axis_name="c", subcore_axis_name="s", num_cores=_NC, num_subcores=_NS
)


@functools.partial(
    pl.kernel,
    out_type=jax.ShapeDtypeStruct((BK, D), jnp.float32),
    mesh=_sc_mesh,
    scratch_types=[
        pltpu.VMEM((_GPW,), jnp.int32),
        pltpu.VMEM((_GPW, D), jnp.float32),
        pltpu.SemaphoreType.DMA,
    ],
)
def _sc_gather(x_hbm, idx_hbm, out_hbm, idx_v, rows_v, sem):
    wid = lax.axis_index("s") * _NC + lax.axis_index("c")
    base = wid * _GPW
    pltpu.sync_copy(idx_hbm.at[pl.ds(base, _GPW)], idx_v)
    pltpu.async_copy(x_hbm.at[idx_v], rows_v, sem).wait()
    pltpu.sync_copy(rows_v, out_hbm.at[pl.ds(base, _GPW)])


@functools.partial(
    pl.kernel,
    out_type=(),
    mesh=_sc_mesh,
    scratch_types=[
        pltpu.VMEM((_GPW,), jnp.int32),
        pltpu.VMEM((_GPW, D), jnp.float32),
        pltpu.SemaphoreType.DMA,
    ],
)
def _sc_scatter(fin_hbm, idx_hbm, out_hbm, idx_v, rows_v, sem):
    wid = lax.axis_index("s") * _NC + lax.axis_index("c")
    base = wid * _GPW
    pltpu.sync_copy(idx_hbm.at[pl.ds(base, _GPW)], idx_v)
    pltpu.sync_copy(fin_hbm.at[pl.ds(base, _GPW)], rows_v)
    pltpu.async_copy(rows_v, out_hbm.at[idx_v], sem).wait()


def _logits_body(x_ref, wa_ref, al_ref, xc_ref):
    xb = x_ref[...]
    al_ref[...] = jnp.dot(
        xb, wa_ref[...][0], precision=jax.lax.Precision.HIGHEST
    )[None, None, :]
    xc_ref[...] = xb


def _logits_call(x_flat, w_aux):
    return pl.pallas_call(
        _logits_body,
        grid=(BS // 512,),
        in_specs=[
            pl.BlockSpec((512, D), lambda i: (i, 0)),
            pl.BlockSpec((1, D), lambda i: (0, 0)),
        ],
        out_specs=[
            pl.BlockSpec((1, 1, 512), lambda i: (i, 0, 0)),
            pl.BlockSpec((512, D), lambda i: (i, 0)),
        ],
        out_shape=[
            jax.ShapeDtypeStruct((BS // 512, 1, 512), jnp.float32),
            jax.ShapeDtypeStruct((BS, D), jnp.float32),
        ],
    )(x_flat, w_aux)


def _rms(x, grow):
    return x * lax.rsqrt(jnp.mean(x * x, axis=-1, keepdims=True) + EPS) * grow


def _attn_body(filt_ref, g1_ref, wq_ref, wk_ref, wv_ref, o_ref, hn_s, cs_s):
    h = pl.program_id(0)
    half = HD // 2
    scale = 1.0 / math.sqrt(HD)

    @pl.when(h == 0)
    def _():
        filt = filt_ref[...]
        hn_s[...] = _rms(filt, g1_ref[...][0]).astype(jnp.bfloat16)
        posk = lax.broadcasted_iota(jnp.int32, (BK, half), 0) % KP
        pos = posk.astype(jnp.float32)
        fidx = lax.broadcasted_iota(jnp.int32, (BK, half), 1).astype(jnp.float32)
        ang = pos * jnp.exp(fidx * (-math.log(10000.0) / half))
        cs_s[...] = jnp.concatenate([jnp.cos(ang), jnp.sin(ang)], axis=1)

    hn = hn_s[...]
    f32 = jnp.float32
    bf16 = jnp.bfloat16
    q = jnp.dot(hn, wq_ref[...].astype(bf16), preferred_element_type=f32)
    k = jnp.dot(hn, wk_ref[...].astype(bf16), preferred_element_type=f32)
    v = jnp.dot(hn, wv_ref[...].astype(bf16), preferred_element_type=f32)
    cs = cs_s[...]
    cosv, sinv = cs[:, :half], cs[:, half:]

    def rope(t):
        t1, t2 = t[:, :half], t[:, half:]
        return jnp.concatenate([t1 * cosv - t2 * sinv, t1 * sinv + t2 * cosv], axis=1)

    q = rope(q)
    k = rope(k)
    ri = lax.broadcasted_iota(jnp.int32, (KP, KP), 0)
    ci = lax.broadcasted_iota(jnp.int32, (KP, KP), 1)
    allowed = (ci <= ri) & (ci < K)
    outs = []
    for b in range(B):
        qb = q[b * KP:(b + 1) * KP].astype(bf16)
        kb = k[b * KP:(b + 1) * KP].astype(bf16)
        vb = v[b * KP:(b + 1) * KP].astype(bf16)
        att = lax.dot_general(
            qb, kb, (((1,), (1,)), ((), ())), preferred_element_type=f32
        ) * scale
        att = jnp.where(allowed, att, NEG)
        att = jax.nn.softmax(att, axis=-1).astype(bf16)
        outs.append(jnp.dot(att, vb, preferred_element_type=f32))
    o_ref[...] = jnp.concatenate(outs, axis=0).astype(bf16)


def _attn_call(filt_flat, g1r, wq, wk, wv):
    return pl.pallas_call(
        _attn_body,
        grid=(H,),
        in_specs=[
            pl.BlockSpec((BK, D), lambda h: (0, 0)),
            pl.BlockSpec((1, D), lambda h: (0, 0)),
            pl.BlockSpec((D, HD), lambda h: (0, h)),
            pl.BlockSpec((D, HD), lambda h: (0, h)),
            pl.BlockSpec((D, HD), lambda h: (0, h)),
        ],
        out_specs=pl.BlockSpec((BK, HD), lambda h: (0, h)),
        out_shape=jax.ShapeDtypeStruct((BK, H * HD), jnp.bfloat16),
        scratch_shapes=[
            pltpu.VMEM((BK, D), jnp.bfloat16),
            pltpu.VMEM((BK, HD), jnp.float32),
        ],
    )(filt_flat, g1r, wq, wk, wv)


def _proj_body(filt_ref, o_ref, wo_ref, out_ref):
    out_ref[...] = filt_ref[...] + jnp.dot(
        o_ref[...], wo_ref[...].astype(jnp.bfloat16),
        preferred_element_type=jnp.float32,
    )


def _proj_call(filt_flat, o_all, wo):
    return pl.pallas_call(
        _proj_body,
        grid=(1,),
        in_specs=[
            pl.BlockSpec((BK, D), lambda i: (0, 0)),
            pl.BlockSpec((BK, H * HD), lambda i: (0, 0)),
            pl.BlockSpec((H * HD, D), lambda i: (0, 0)),
        ],
        out_specs=pl.BlockSpec((BK, D), lambda i: (0, 0)),
        out_shape=jax.ShapeDtypeStruct((BK, D), jnp.float32),
    )(filt_flat, o_all, wo)


_TH = 256
_NT = HIDDEN // _TH


def _ffn_body(attn_ref, filt_ref, w_ref, g2_ref, w1_ref, w3_ref, w2_ref, out_ref):
    j = pl.program_id(0)
    a = attn_ref[...]
    f32 = jnp.float32
    bf16 = jnp.bfloat16
    h2 = _rms(a, g2_ref[...][0]).astype(bf16)
    u = jnp.dot(h2, w1_ref[...].astype(bf16), preferred_element_type=f32)
    t3 = jnp.dot(h2, w3_ref[...].astype(bf16), preferred_element_type=f32)
    gg = ((u * jax.nn.sigmoid(u)) * t3).astype(bf16)
    contrib = jnp.dot(gg, w2_ref[...].astype(bf16), preferred_element_type=f32)

    @pl.when(j == 0)
    def _():
        out_ref[...] = a + contrib

    @pl.when((j > 0) & (j < _NT - 1))
    def _():
        out_ref[...] += contrib

    @pl.when(j == _NT - 1)
    def _():
        full = out_ref[...] + contrib
        fin = filt_ref[...] + w_ref[...][:, :1] * full
        fin4 = fin.reshape(B, KP, D)
        row_last = fin4[:, K - 1 : K, :]
        rid = lax.broadcasted_iota(jnp.int32, (B, KP, D), 1)
        fin4 = jnp.where(rid >= K, row_last, fin4)
        out_ref[...] = fin4.reshape(BK, D)


def _ffn_call(attn_flat, filt_flat, wbc, g2r, w1, w3, w2):
    return pl.pallas_call(
        _ffn_body,
        grid=(_NT,),
        in_specs=[
            pl.BlockSpec((BK, D), lambda j: (0, 0)),
            pl.BlockSpec((BK, D), lambda j: (0, 0)),
            pl.BlockSpec((BK, 128), lambda j: (0, 0)),
            pl.BlockSpec((1, D), lambda j: (0, 0)),
            pl.BlockSpec((D, _TH), lambda j: (0, j)),
            pl.BlockSpec((D, _TH), lambda j: (0, j)),
            pl.BlockSpec((_TH, D), lambda j: (j, 0)),
        ],
        out_specs=pl.BlockSpec((BK, D), lambda j: (0, 0)),
        out_shape=jax.ShapeDtypeStruct((BK, D), jnp.float32),
    )(attn_flat, filt_flat, wbc, g2r, w1, w3, w2)


def _aux_body(al_ref, t_ref, out_ref):
    al = al_ref[...]
    t = t_ref[...]
    p = jnp.clip(jax.nn.sigmoid(al), 1e-7, 1.0 - 1e-7)
    loss = t * jnp.log(p) + (1.0 - t) * jnp.log(1.0 - p)
    out_ref[...] = jnp.broadcast_to(-jnp.mean(loss), (1, 1))


def _aux_call(al16, t16):
    return pl.pallas_call(
        _aux_body,
        grid=(1,),
        in_specs=[
            pl.BlockSpec((BS // 512, 512), lambda i: (0, 0)),
            pl.BlockSpec((BS // 512, 512), lambda i: (0, 0)),
        ],
        out_specs=pl.BlockSpec((1, 1), lambda i: (0, 0)),
        out_shape=jax.ShapeDtypeStruct((1, 1), jnp.float32),
    )(al16, t16)


def kernel(x, W_router, W_aux, g1, g2, Wq, Wk, Wv, Wo, W1, W2, W3):
    x_flat = x.reshape(BS, D)
    al16, xcopy = _logits_call(x_flat, W_aux)
    al16 = al16.reshape(BS // 512, 512)
    rl = (x @ W_router.T)[..., 0]

    token_weights, token_index = jax.lax.top_k(rl, K)
    order = jnp.argsort(token_index, axis=1)
    sel = jnp.take_along_axis(token_index, order, axis=1)

    sel_pad = jnp.concatenate(
        [sel, jnp.broadcast_to(sel[:, K - 1 : K], (B, NPAD))], axis=1
    )
    gidx = (
        sel_pad + (jnp.arange(B, dtype=sel_pad.dtype) * S)[:, None]
    ).reshape(BK).astype(jnp.int32)

    filtered = _sc_gather(x_flat, gidx)
    o_all = _attn_call(filtered, g1.reshape(1, D), Wq, Wk, Wv)
    attn = _proj_call(filtered, o_all, Wo)

    twpad = jnp.concatenate(
        [token_weights, jnp.zeros((B, NPAD), token_weights.dtype)], axis=1
    ).reshape(BK)
    wbc = jnp.broadcast_to(twpad[:, None], (BK, 128))
    fin = _ffn_call(
        attn, filtered, wbc, g2.reshape(1, D), W1, W3, W2
    )

    out_ref = jax.new_ref(xcopy)
    _sc_scatter(fin, gidx, out_ref)
    out = out_ref[...].reshape(B, S, D)

    t_vec = jnp.zeros((BS,), jnp.float32).at[sel.reshape(-1)].set(1.0)
    aux = _aux_call(al16, t_vec.reshape(BS // 512, 512))[0, 0]
    return out, aux


# speedup vs baseline: 1.0331x; 1.0331x over previous
import functools
import math

import jax
import jax.numpy as jnp
from jax import lax
from jax.experimental import pallas as pl
from jax.experimental.pallas import tpu as pltpu
from jax.experimental.pallas import tpu_sc as plsc

B, S, D = 4, 2048, 2048
H, HD = 16, 128
HIDDEN = 5632
CAP = 0.12
EPS = 1e-5
K = int(S * CAP)
KP = 256
NPAD = KP - K
BS = B * S
BK = B * KP
NEG = float(jnp.finfo(jnp.float32).min)

_NC, _NS = 2, 16
_NW = _NC * _NS
_GPW = BK // _NW
_CPW = BS // _NW

_sc_mesh = plsc.VectorSubcoreMesh(
    core_axis_name="c", subcore_axis_name="s", num_cores=_NC, num_subcores=_NS
)


@functools.partial(
    pl.kernel,
    out_type=jax.ShapeDtypeStruct((BK, D), jnp.float32),
    mesh=_sc_mesh,
    scratch_types=[
        pltpu.VMEM((_GPW,), jnp.int32),
        pltpu.VMEM((_GPW, D), jnp.float32),
        pltpu.SemaphoreType.DMA,
    ],
)
def _sc_gather(x_hbm, idx_hbm, out_hbm, idx_v, rows_v, sem):
    wid = lax.axis_index("s") * _NC + lax.axis_index("c")
    base = wid * _GPW
    pltpu.sync_copy(idx_hbm.at[pl.ds(base, _GPW)], idx_v)
    pltpu.async_copy(x_hbm.at[idx_v], rows_v, sem).wait()
    pltpu.sync_copy(rows_v, out_hbm.at[pl.ds(base, _GPW)])


@functools.partial(
    pl.kernel,
    out_type=(),
    mesh=_sc_mesh,
    scratch_types=[
        pltpu.VMEM((_GPW,), jnp.int32),
        pltpu.VMEM((_GPW, D), jnp.float32),
        pltpu.SemaphoreType.DMA,
    ],
)
def _sc_scatter(fin_hbm, idx_hbm, out_hbm, idx_v, rows_v, sem):
    wid = lax.axis_index("s") * _NC + lax.axis_index("c")
    base = wid * _GPW
    pltpu.sync_copy(idx_hbm.at[pl.ds(base, _GPW)], idx_v)
    pltpu.sync_copy(fin_hbm.at[pl.ds(base, _GPW)], rows_v)
    pltpu.async_copy(rows_v, out_hbm.at[idx_v], sem).wait()


def _logits_body(x_ref, wa_ref, al_ref, xc_ref):
    xb = x_ref[...]
    al_ref[...] = jnp.dot(
        xb, wa_ref[...][0], precision=jax.lax.Precision.HIGHEST
    )[None, None, :]
    xc_ref[...] = xb


def _logits_call(x_flat, w_aux):
    return pl.pallas_call(
        _logits_body,
        grid=(BS // 512,),
        in_specs=[
            pl.BlockSpec((512, D), lambda i: (i, 0)),
            pl.BlockSpec((1, D), lambda i: (0, 0)),
        ],
        out_specs=[
            pl.BlockSpec((1, 1, 512), lambda i: (i, 0, 0)),
            pl.BlockSpec((512, D), lambda i: (i, 0)),
        ],
        out_shape=[
            jax.ShapeDtypeStruct((BS // 512, 1, 512), jnp.float32),
            jax.ShapeDtypeStruct((BS, D), jnp.float32),
        ],
    )(x_flat, w_aux)


def _rms(x, grow):
    return x * lax.rsqrt(jnp.mean(x * x, axis=-1, keepdims=True) + EPS) * grow


def _attn_body(filt_ref, g1_ref, wq_ref, wk_ref, wv_ref, o_ref, hn_s, cs_s):
    h = pl.program_id(0)
    half = HD // 2
    scale = 1.0 / math.sqrt(HD)

    @pl.when(h == 0)
    def _():
        filt = filt_ref[...]
        hn_s[...] = _rms(filt, g1_ref[...][0])
        posk = lax.broadcasted_iota(jnp.int32, (BK, half), 0) % KP
        pos = posk.astype(jnp.float32)
        fidx = lax.broadcasted_iota(jnp.int32, (BK, half), 1).astype(jnp.float32)
        ang = pos * jnp.exp(fidx * (-math.log(10000.0) / half))
        cs_s[...] = jnp.concatenate([jnp.cos(ang), jnp.sin(ang)], axis=1)

    hn = hn_s[...]
    q = jnp.dot(hn, wq_ref[...])
    k = jnp.dot(hn, wk_ref[...])
    v = jnp.dot(hn, wv_ref[...])
    cs = cs_s[...]
    cosv, sinv = cs[:, :half], cs[:, half:]

    def rope(t):
        t1, t2 = t[:, :half], t[:, half:]
        return jnp.concatenate([t1 * cosv - t2 * sinv, t1 * sinv + t2 * cosv], axis=1)

    q = rope(q)
    k = rope(k)
    ri = lax.broadcasted_iota(jnp.int32, (KP, KP), 0)
    ci = lax.broadcasted_iota(jnp.int32, (KP, KP), 1)
    allowed = (ci <= ri) & (ci < K)
    outs = []
    for b in range(B):
        qb = q[b * KP:(b + 1) * KP]
        kb = k[b * KP:(b + 1) * KP]
        vb = v[b * KP:(b + 1) * KP]
        att = lax.dot_general(qb, kb, (((1,), (1,)), ((), ()))) * scale
        att = jnp.where(allowed, att, NEG)
        att = jax.nn.softmax(att, axis=-1)
        outs.append(jnp.dot(att, vb))
    o_ref[...] = jnp.concatenate(outs, axis=0)


def _attn_call(filt_flat, g1r, wq, wk, wv):
    return pl.pallas_call(
        _attn_body,
        grid=(H,),
        in_specs=[
            pl.BlockSpec((BK, D), lambda h: (0, 0)),
            pl.BlockSpec((1, D), lambda h: (0, 0)),
            pl.BlockSpec((D, HD), lambda h: (0, h)),
            pl.BlockSpec((D, HD), lambda h: (0, h)),
            pl.BlockSpec((D, HD), lambda h: (0, h)),
        ],
        out_specs=pl.BlockSpec((BK, HD), lambda h: (0, h)),
        out_shape=jax.ShapeDtypeStruct((BK, H * HD), jnp.float32),
        scratch_shapes=[
            pltpu.VMEM((BK, D), jnp.float32),
            pltpu.VMEM((BK, HD), jnp.float32),
        ],
    )(filt_flat, g1r, wq, wk, wv)


def _proj_body(filt_ref, o_ref, wo_ref, out_ref):
    out_ref[...] = filt_ref[...] + jnp.dot(o_ref[...], wo_ref[...])


def _proj_call(filt_flat, o_all, wo):
    return pl.pallas_call(
        _proj_body,
        grid=(1,),
        in_specs=[
            pl.BlockSpec((BK, D), lambda i: (0, 0)),
            pl.BlockSpec((BK, H * HD), lambda i: (0, 0)),
            pl.BlockSpec((H * HD, D), lambda i: (0, 0)),
        ],
        out_specs=pl.BlockSpec((BK, D), lambda i: (0, 0)),
        out_shape=jax.ShapeDtypeStruct((BK, D), jnp.float32),
    )(filt_flat, o_all, wo)


_TH = 256
_NT = HIDDEN // _TH


def _ffn_body(attn_ref, filt_ref, w_ref, g2_ref, w1_ref, w3_ref, w2_ref, out_ref, h2_s):
    j = pl.program_id(0)
    f32 = jnp.float32
    bf16 = jnp.bfloat16

    @pl.when(j == 0)
    def _():
        h2_s[...] = _rms(attn_ref[...], g2_ref[...][0]).astype(bf16)

    h2 = h2_s[...]
    u = jnp.dot(h2, w1_ref[...].astype(bf16), preferred_element_type=f32)
    t3 = jnp.dot(h2, w3_ref[...].astype(bf16), preferred_element_type=f32)
    gg = ((u * jax.nn.sigmoid(u)) * t3).astype(bf16)
    contrib = jnp.dot(gg, w2_ref[...].astype(bf16), preferred_element_type=f32)

    @pl.when(j == 0)
    def _():
        out_ref[...] = attn_ref[...] + contrib

    @pl.when((j > 0) & (j < _NT - 1))
    def _():
        out_ref[...] += contrib

    @pl.when(j == _NT - 1)
    def _():
        full = out_ref[...] + contrib
        fin = filt_ref[...] + w_ref[...][:, :1] * full
        fin4 = fin.reshape(B, KP, D)
        row_last = fin4[:, K - 1 : K, :]
        rid = lax.broadcasted_iota(jnp.int32, (B, KP, D), 1)
        fin4 = jnp.where(rid >= K, row_last, fin4)
        out_ref[...] = fin4.reshape(BK, D)


def _ffn_call(attn_flat, filt_flat, wbc, g2r, w1, w3, w2):
    return pl.pallas_call(
        _ffn_body,
        grid=(_NT,),
        in_specs=[
            pl.BlockSpec((BK, D), lambda j: (0, 0)),
            pl.BlockSpec((BK, D), lambda j: (0, 0)),
            pl.BlockSpec((BK, 128), lambda j: (0, 0)),
            pl.BlockSpec((1, D), lambda j: (0, 0)),
            pl.BlockSpec((D, _TH), lambda j: (0, j)),
            pl.BlockSpec((D, _TH), lambda j: (0, j)),
            pl.BlockSpec((_TH, D), lambda j: (j, 0)),
        ],
        out_specs=pl.BlockSpec((BK, D), lambda j: (0, 0)),
        out_shape=jax.ShapeDtypeStruct((BK, D), jnp.float32),
        scratch_shapes=[pltpu.VMEM((BK, D), jnp.bfloat16)],
    )(attn_flat, filt_flat, wbc, g2r, w1, w3, w2)


def _aux_body(al_ref, t_ref, out_ref):
    al = al_ref[...]
    t = t_ref[...]
    p = jnp.clip(jax.nn.sigmoid(al), 1e-7, 1.0 - 1e-7)
    loss = t * jnp.log(p) + (1.0 - t) * jnp.log(1.0 - p)
    out_ref[...] = jnp.broadcast_to(-jnp.mean(loss), (1, 1))


def _aux_call(al16, t16):
    return pl.pallas_call(
        _aux_body,
        grid=(1,),
        in_specs=[
            pl.BlockSpec((BS // 512, 512), lambda i: (0, 0)),
            pl.BlockSpec((BS // 512, 512), lambda i: (0, 0)),
        ],
        out_specs=pl.BlockSpec((1, 1), lambda i: (0, 0)),
        out_shape=jax.ShapeDtypeStruct((1, 1), jnp.float32),
    )(al16, t16)


def kernel(x, W_router, W_aux, g1, g2, Wq, Wk, Wv, Wo, W1, W2, W3):
    x_flat = x.reshape(BS, D)
    al16, xcopy = _logits_call(x_flat, W_aux)
    al16 = al16.reshape(BS // 512, 512)
    rl = (x @ W_router.T)[..., 0]

    token_weights, token_index = jax.lax.top_k(rl, K)
    order = jnp.argsort(token_index, axis=1)
    sel = jnp.take_along_axis(token_index, order, axis=1)

    sel_pad = jnp.concatenate(
        [sel, jnp.broadcast_to(sel[:, K - 1 : K], (B, NPAD))], axis=1
    )
    gidx = (
        sel_pad + (jnp.arange(B, dtype=sel_pad.dtype) * S)[:, None]
    ).reshape(BK).astype(jnp.int32)

    filtered = _sc_gather(x_flat, gidx)
    o_all = _attn_call(filtered, g1.reshape(1, D), Wq, Wk, Wv)
    attn = _proj_call(filtered, o_all, Wo)

    twpad = jnp.concatenate(
        [token_weights, jnp.zeros((B, NPAD), token_weights.dtype)], axis=1
    ).reshape(BK)
    wbc = jnp.broadcast_to(twpad[:, None], (BK, 128))
    fin = _ffn_call(
        attn, filtered, wbc, g2.reshape(1, D), W1, W3, W2
    )

    out_ref = jax.new_ref(xcopy)
    _sc_scatter(fin, gidx, out_ref)
    out = out_ref[...].reshape(B, S, D)

    t_vec = jnp.zeros((BS,), jnp.float32).at[sel.reshape(-1)].set(1.0)
    aux = _aux_call(al16, t_vec.reshape(BS // 512, 512))[0, 0]
    return out, aux
